# baseline (speedup 1.0000x reference)
import functools

import jax
import jax.numpy as jnp
from jax import lax
from jax.experimental import pallas as pl
from jax.experimental.pallas import tpu as pltpu
from jax.experimental.pallas import tpu_sc as plsc

EMB = 128
NQ = 5000
R = 5120
NC = 2
NS = 16
NW = NC * NS
CH = 128
BLK = 512


NB = 2


def _sc_aggregate(node_features, il):
    nbt = il.shape[0]
    nj = nbt // NW
    stripe = R // NS

    mesh = plsc.VectorSubcoreMesh(
        core_axis_name="c", subcore_axis_name="s",
        num_cores=NC, num_subcores=NS)

    @functools.partial(
        pl.kernel,
        out_type=(
            jax.ShapeDtypeStruct((NC, R, EMB), jnp.float32),
            jax.ShapeDtypeStruct((NC * R,), jnp.float32),
        ),
        mesh=mesh,
        scratch_types=[
            pltpu.VMEM((NB, 2, CH), jnp.int32),
            pltpu.VMEM((NB, 2, CH), jnp.int32),
            pltpu.VMEM((NB * CH, EMB), jnp.float32),
            pltpu.VMEM((NB * CH, EMB), jnp.float32),
            pltpu.VMEM((CH,), jnp.float32),
            pltpu.VMEM_SHARED((R, EMB), jnp.float32),
            pltpu.VMEM_SHARED((R,), jnp.float32),
            pltpu.SemaphoreType.DMA,
        ],
    )
    def k(nf_hbm, il_hbm, agg_out, deg_out,
          idx_a, idx_b, rows_a, rows_b, ones_v, agg_sh, deg_sh, gsem):
        c_idx = lax.axis_index("c")
        s_idx = lax.axis_index("s")
        wid = s_idx * NC + c_idx

        zero16 = jnp.zeros((16,), jnp.float32)
        one16 = jnp.ones((16,), jnp.float32)
        lane16 = lax.iota(jnp.int32, 16)

        def zrow(r, carry):
            row = rows_a.at[r]
            for i in range(EMB // 16):
                row[pl.ds(i * 16, 16)] = zero16
            return carry
        lax.fori_loop(0, CH, zrow, 0)
        for i in range(CH // 16):
            ones_v[pl.ds(i * 16, 16)] = one16

        base = s_idx * stripe
        off = 0
        while off < stripe:
            n = min(CH, stripe - off)
            pltpu.sync_copy(rows_a.at[pl.ds(0, n)],
                            agg_sh.at[pl.ds(base + off, n)])
            off += n
        off = 0
        while off < stripe:
            n = min(EMB, stripe - off)
            pltpu.sync_copy(rows_a.at[0, pl.ds(0, n)],
                            deg_sh.at[pl.ds(base + off, n)])
            off += n
        plsc.subcore_barrier()

        def stage(t, idx_x):
            pltpu.sync_copy(il_hbm.at[wid + NW * t], idx_x)
            for b in range(NB):
                dst_row = idx_x.at[b, 1]
                for i in range(CH // 16):
                    d = dst_row[pl.ds(i * 16, 16)]
                    dc = jnp.where(d < NQ, d, NQ + lane16)
                    dst_row[pl.ds(i * 16, 16)] = dc

        def issue(idx_x, rows_x):
            for b in range(NB):
                pltpu.async_copy(nf_hbm.at[idx_x.at[b, 0]],
                                 rows_x.at[pl.ds(b * CH, CH)], gsem)

        def drain(rows_x):
            pltpu.make_async_copy(nf_hbm.at[pl.ds(0, NB * CH)],
                                  rows_x, gsem).wait()

        def scatter(idx_x, rows_x):
            for b in range(NB):
                pltpu.sync_copy(rows_x.at[pl.ds(b * CH, CH)],
                                agg_sh.at[idx_x.at[b, 1]], add=True)
                pltpu.sync_copy(ones_v, deg_sh.at[idx_x.at[b, 1]],
                                add=True)

        stage(0, idx_a)
        issue(idx_a, rows_a)

        def body(u, carry):
            t0 = 2 * u
            stage(t0 + 1, idx_b)
            drain(rows_a)
            issue(idx_b, rows_b)
            scatter(idx_a, rows_a)

            @pl.when(u + 1 < nj // 2)
            def _():
                stage(t0 + 2, idx_a)
            drain(rows_b)

            @pl.when(u + 1 < nj // 2)
            def _():
                issue(idx_a, rows_a)
            scatter(idx_b, rows_b)
            return carry

        lax.fori_loop(0, nj // 2, body, 0)
        plsc.subcore_barrier()

        pltpu.sync_copy(agg_sh.at[pl.ds(base, stripe)],
                        agg_out.at[c_idx, pl.ds(base, stripe)])

        @pl.when(s_idx == 0)
        def _():
            pltpu.sync_copy(deg_sh, deg_out.at[pl.ds(c_idx * R, R)])

    return k(node_features, il)


def _tc_dense_body(q_ref, nf_ref, agg_ref, deg_ref, w_ref, cb_ref, ib_ref,
                   wrong_ref, right_ref):
    i = pl.program_id(0)
    agg = agg_ref[0] + agg_ref[1]
    deg = jnp.sum(deg_ref[...], axis=0)
    x = nf_ref[...] + agg / jnp.maximum(deg, 1.0)[:, None]
    h = jnp.maximum(jnp.dot(x, w_ref[...],
                            preferred_element_type=jnp.float32), 0.0)
    rows = i * BLK + lax.broadcasted_iota(jnp.int32, (BLK, EMB), 0)
    base = jnp.where(rows < q_ref[0, 0], h, 0.0)
    wrong_ref[...] = base + ib_ref[...]
    right_ref[...] = base + cb_ref[...]


def _tc_dense(q, node_features, agg_p, deg_p, W, correct_bias, incorrect_bias):
    grid = (NQ + BLK - 1) // BLK
    return pl.pallas_call(
        _tc_dense_body,
        grid=(grid,),
        in_specs=[
            pl.BlockSpec(memory_space=pltpu.SMEM),
            pl.BlockSpec((BLK, EMB), lambda i: (i, 0)),
            pl.BlockSpec((NC, BLK, EMB), lambda i: (0, i, 0)),
            pl.BlockSpec((NC, BLK), lambda i: (0, i)),
            pl.BlockSpec((EMB, EMB), lambda i: (0, 0)),
            pl.BlockSpec((1, EMB), lambda i: (0, 0)),
            pl.BlockSpec((1, EMB), lambda i: (0, 0)),
        ],
        out_specs=[
            pl.BlockSpec((BLK, EMB), lambda i: (i, 0)),
            pl.BlockSpec((BLK, EMB), lambda i: (i, 0)),
        ],
        out_shape=[
            jax.ShapeDtypeStruct((NQ, EMB), jnp.float32),
            jax.ShapeDtypeStruct((NQ, EMB), jnp.float32),
        ],
    )(q, node_features, agg_p, deg_p, W, correct_bias, incorrect_bias)


def kernel(node_features, edge_index, W, correct_bias, incorrect_bias, Q):
    e = edge_index.shape[1]
    ncht = e // CH
    nbt_raw = (ncht + NB - 1) // NB
    nj = (nbt_raw + NW - 1) // NW
    nj = nj + (nj % 2)
    ncht_pad = nj * NW * NB
    src2d = edge_index[0].reshape(ncht, CH)
    dst2d = edge_index[1].reshape(ncht, CH)
    if ncht_pad != ncht:
        padn = ncht_pad - ncht
        src2d = jnp.pad(src2d, ((0, padn), (0, 0)))
        dst2d = jnp.pad(dst2d, ((0, padn), (0, 0)), constant_values=NQ)
    il = jnp.stack([src2d, dst2d], axis=1).reshape(ncht_pad // NB, NB, 2, CH)
    agg_p, deg_p = _sc_aggregate(node_features, il)
    deg_p = deg_p.reshape(NC, R)
    q_arr = jnp.asarray(Q, dtype=jnp.int32).reshape(1, 1)
    wrong, right = _tc_dense(q_arr, node_features, agg_p, deg_p, W,
                             correct_bias, incorrect_bias)
    padding = jnp.zeros((1, EMB), dtype=wrong.dtype)
    return jnp.concatenate([wrong, right, padding], axis=0)


# speedup vs baseline: 2.2579x; 2.2579x over previous
import functools

import jax
import jax.numpy as jnp
from jax import lax
from jax.experimental import pallas as pl
from jax.experimental.pallas import tpu as pltpu
from jax.experimental.pallas import tpu_sc as plsc

EMB = 128
NQ = 5000
R = 5120
NC = 2
NS = 16
NW = NC * NS
CH = 128
BLK = 512


NB = 4


def _sc_aggregate(node_features, il):
    nbt = il.shape[0]
    nj = (nbt + NW - 1) // NW
    stripe = R // NS

    mesh = plsc.VectorSubcoreMesh(
        core_axis_name="c", subcore_axis_name="s",
        num_cores=NC, num_subcores=NS)

    @functools.partial(
        pl.kernel,
        out_type=(
            jax.ShapeDtypeStruct((NC, R, EMB), jnp.float32),
            jax.ShapeDtypeStruct((NC * R,), jnp.float32),
        ),
        mesh=mesh,
        scratch_types=[
            pltpu.VMEM((NB, 2, CH), jnp.int32),
            pltpu.VMEM((NB * CH, EMB), jnp.float32),
            pltpu.VMEM((CH,), jnp.float32),
            pltpu.VMEM_SHARED((R, EMB), jnp.float32),
            pltpu.VMEM_SHARED((R,), jnp.float32),
            pltpu.SemaphoreType.DMA,
        ],
    )
    def k(nf_hbm, il_hbm, agg_out, deg_out,
          idx_v, rows_v, ones_v, agg_sh, deg_sh, gsem):
        c_idx = lax.axis_index("c")
        s_idx = lax.axis_index("s")
        wid = s_idx * NC + c_idx

        zero16 = jnp.zeros((16,), jnp.float32)
        one16 = jnp.ones((16,), jnp.float32)
        lane16 = lax.iota(jnp.int32, 16)

        def zrow(r, carry):
            row = rows_v.at[r]
            for i in range(EMB // 16):
                row[pl.ds(i * 16, 16)] = zero16
            return carry
        lax.fori_loop(0, CH, zrow, 0)
        for i in range(CH // 16):
            ones_v[pl.ds(i * 16, 16)] = one16

        base = s_idx * stripe
        off = 0
        while off < stripe:
            n = min(CH, stripe - off)
            pltpu.sync_copy(rows_v.at[pl.ds(0, n)],
                            agg_sh.at[pl.ds(base + off, n)])
            off += n
        off = 0
        while off < stripe:
            n = min(EMB, stripe - off)
            pltpu.sync_copy(rows_v.at[0, pl.ds(0, n)],
                            deg_sh.at[pl.ds(base + off, n)])
            off += n
        plsc.subcore_barrier()

        def body(j, carry):
            bt = wid + NW * j

            @pl.when(bt < nbt)
            def _():
                pltpu.sync_copy(il_hbm.at[bt], idx_v)
                for b in range(NB):
                    pltpu.async_copy(nf_hbm.at[idx_v.at[b, 0]],
                                     rows_v.at[pl.ds(b * CH, CH)], gsem)
                for b in range(NB):
                    dst_row = idx_v.at[b, 1]
                    for i in range(CH // 16):
                        d = dst_row[pl.ds(i * 16, 16)]
                        dc = jnp.where(d < NQ, d, NQ + lane16)
                        dst_row[pl.ds(i * 16, 16)] = dc
                pltpu.make_async_copy(nf_hbm.at[pl.ds(0, NB * CH)],
                                      rows_v, gsem).wait()
                for b in range(NB):
                    pltpu.sync_copy(rows_v.at[pl.ds(b * CH, CH)],
                                    agg_sh.at[idx_v.at[b, 1]], add=True)
                    pltpu.sync_copy(ones_v, deg_sh.at[idx_v.at[b, 1]],
                                    add=True)
            return carry

        lax.fori_loop(0, nj, body, 0)
        plsc.subcore_barrier()

        pltpu.sync_copy(agg_sh.at[pl.ds(base, stripe)],
                        agg_out.at[c_idx, pl.ds(base, stripe)])

        @pl.when(s_idx == 0)
        def _():
            pltpu.sync_copy(deg_sh, deg_out.at[pl.ds(c_idx * R, R)])

    return k(node_features, il)


def _tc_dense_body(q_ref, nf_ref, agg_ref, deg_ref, w_ref, cb_ref, ib_ref,
                   wrong_ref, right_ref):
    i = pl.program_id(0)
    agg = agg_ref[0] + agg_ref[1]
    deg = jnp.sum(deg_ref[...], axis=0)
    x = nf_ref[...] + agg / jnp.maximum(deg, 1.0)[:, None]
    h = jnp.maximum(jnp.dot(x, w_ref[...],
                            preferred_element_type=jnp.float32), 0.0)
    rows = i * BLK + lax.broadcasted_iota(jnp.int32, (BLK, EMB), 0)
    base = jnp.where(rows < q_ref[0, 0], h, 0.0)
    wrong_ref[...] = base + ib_ref[...]
    right_ref[...] = base + cb_ref[...]


def _tc_dense(q, node_features, agg_p, deg_p, W, correct_bias, incorrect_bias):
    grid = (NQ + BLK - 1) // BLK
    return pl.pallas_call(
        _tc_dense_body,
        grid=(grid,),
        in_specs=[
            pl.BlockSpec(memory_space=pltpu.SMEM),
            pl.BlockSpec((BLK, EMB), lambda i: (i, 0)),
            pl.BlockSpec((NC, BLK, EMB), lambda i: (0, i, 0)),
            pl.BlockSpec((NC, BLK), lambda i: (0, i)),
            pl.BlockSpec((EMB, EMB), lambda i: (0, 0)),
            pl.BlockSpec((1, EMB), lambda i: (0, 0)),
            pl.BlockSpec((1, EMB), lambda i: (0, 0)),
        ],
        out_specs=[
            pl.BlockSpec((BLK, EMB), lambda i: (i, 0)),
            pl.BlockSpec((BLK, EMB), lambda i: (i, 0)),
        ],
        out_shape=[
            jax.ShapeDtypeStruct((NQ, EMB), jnp.float32),
            jax.ShapeDtypeStruct((NQ, EMB), jnp.float32),
        ],
    )(q, node_features, agg_p, deg_p, W, correct_bias, incorrect_bias)


def kernel(node_features, edge_index, W, correct_bias, incorrect_bias, Q):
    e = edge_index.shape[1]
    ncht = e // CH
    ncht_pad = ((ncht + NB - 1) // NB) * NB
    src2d = edge_index[0].reshape(ncht, CH)
    dst2d = edge_index[1].reshape(ncht, CH)
    if ncht_pad != ncht:
        padn = ncht_pad - ncht
        src2d = jnp.pad(src2d, ((0, padn), (0, 0)))
        dst2d = jnp.pad(dst2d, ((0, padn), (0, 0)), constant_values=NQ)
    il = jnp.stack([src2d, dst2d], axis=1).reshape(ncht_pad // NB, NB, 2, CH)
    agg_p, deg_p = _sc_aggregate(node_features, il)
    deg_p = deg_p.reshape(NC, R)
    q_arr = jnp.asarray(Q, dtype=jnp.int32).reshape(1, 1)
    wrong, right = _tc_dense(q_arr, node_features, agg_p, deg_p, W,
                             correct_bias, incorrect_bias)
    padding = jnp.zeros((1, EMB), dtype=wrong.dtype)
    return jnp.concatenate([wrong, right, padding], axis=0)
